# Initial kernel scaffold; baseline (speedup 1.0000x reference)
import jax
import jax.numpy as jnp
from jax.experimental import pallas as pl


def kernel(world_pos, prev_world_pos, node_type, mesh_pos, cells, params):
    raise NotImplementedError("write your pallas kernel here")


# speedup vs baseline: 1.0663x; 1.0663x over previous
import functools

import jax
import jax.numpy as jnp
from jax import lax
from jax.experimental import pallas as pl
from jax.experimental.pallas import tpu as pltpu
from jax.experimental.pallas import tpu_sc as plsc

N_NODES = 10000
N_TRIS = 20000
NODE_TYPE_SIZE = 9
LATENT = 128
MP_STEPS = 15

NC, NS = 2, 16
NW = NC * NS
E_DIR = 6 * N_TRIS
CHUNK = 128
EPW = 3840
E_PAD = NW * EPW
N_CHUNKS = EPW // CHUNK
N_PAD = 10240
TRASH = N_NODES
ROWS_PER_SUB = N_PAD // NS
BE = 1024
BN = 1024


@functools.lru_cache(maxsize=None)
def _make_gather2(n_rows_out, d):
    epw = n_rows_out // NW
    nchunks = epw // CHUNK
    mesh = plsc.VectorSubcoreMesh(core_axis_name="c", subcore_axis_name="s")

    @functools.partial(
        pl.kernel,
        out_type=(jax.ShapeDtypeStruct((n_rows_out, d), jnp.float32),
                  jax.ShapeDtypeStruct((n_rows_out, d), jnp.float32)),
        mesh=mesh,
        scratch_types=[
            pltpu.VMEM((CHUNK,), jnp.int32),
            pltpu.VMEM((CHUNK,), jnp.int32),
            pltpu.VMEM((CHUNK, d), jnp.float32),
            pltpu.VMEM((CHUNK, d), jnp.float32),
            pltpu.SemaphoreType.DMA,
            pltpu.SemaphoreType.DMA,
        ],
    )
    def gather2(table, idx_s, idx_r, out_s, out_r,
                idxv_s, idxv_r, buf_s, buf_r, sem_s, sem_r):
        wid = lax.axis_index("s") * NC + lax.axis_index("c")
        base = wid * epw

        def body(j, carry):
            off = base + j * CHUNK
            pltpu.sync_copy(idx_s.at[pl.ds(off, CHUNK)], idxv_s)
            pltpu.sync_copy(idx_r.at[pl.ds(off, CHUNK)], idxv_r)
            a = pltpu.async_copy(table.at[idxv_s], buf_s, sem_s)
            b = pltpu.async_copy(table.at[idxv_r], buf_r, sem_r)
            a.wait()
            b.wait()
            pltpu.sync_copy(buf_s, out_s.at[pl.ds(off, CHUNK)])
            pltpu.sync_copy(buf_r, out_r.at[pl.ds(off, CHUNK)])
            return carry

        lax.fori_loop(0, nchunks, body, 0)

    return gather2


@functools.lru_cache(maxsize=None)
def _make_scatter():
    mesh = plsc.VectorSubcoreMesh(core_axis_name="c", subcore_axis_name="s")

    @functools.partial(
        pl.kernel,
        out_type=jax.ShapeDtypeStruct((NC, N_PAD, LATENT), jnp.float32),
        mesh=mesh,
        scratch_types=[
            pltpu.VMEM((CHUNK,), jnp.int32),
            pltpu.VMEM((CHUNK, LATENT), jnp.float32),
            pltpu.VMEM_SHARED((N_PAD, LATENT), jnp.float32),
        ],
    )
    def scatter(edges, ridx, zeros, out, idxv, buf, acc):
        cid = lax.axis_index("c")
        sid = lax.axis_index("s")
        wid = sid * NC + cid
        pltpu.sync_copy(zeros, acc.at[pl.ds(sid * ROWS_PER_SUB, ROWS_PER_SUB)])
        plsc.subcore_barrier()
        base = wid * EPW

        def body(j, carry):
            off = base + j * CHUNK
            pltpu.sync_copy(ridx.at[pl.ds(off, CHUNK)], idxv)
            pltpu.sync_copy(edges.at[pl.ds(off, CHUNK)], buf)
            pltpu.sync_copy(buf, acc.at[idxv], add=True)
            return carry

        lax.fori_loop(0, N_CHUNKS, body, 0)
        plsc.subcore_barrier()
        pltpu.sync_copy(acc.at[pl.ds(sid * ROWS_PER_SUB, ROWS_PER_SUB)],
                        out.at[cid].at[pl.ds(sid * ROWS_PER_SUB, ROWS_PER_SUB)])

    return scatter


def _gather2_call(table, idx_s, idx_r):
    return _make_gather2(int(idx_s.shape[0]), int(table.shape[1]))(table, idx_s, idx_r)


def _scatter_call(edges, ridx, zeros):
    return _make_scatter()(edges, ridx, zeros)


def _ln_res(x, y, g, b):
    mu = jnp.mean(y, axis=-1, keepdims=True)
    var = jnp.mean((y - mu) ** 2, axis=-1, keepdims=True)
    return x + (y - mu) * lax.rsqrt(var + 1e-5) * g + b


def _dot(a, b):
    return jnp.dot(a, b, preferred_element_type=jnp.float32,
                   precision=lax.Precision.HIGHEST)


def _edge_step_body(el_ref, gs_ref, gr_ref, w1e, w1s, w1r, b1, w2, b2, w3, b3,
                    g, bln, out_ref):
    x = el_ref[...]
    h = _dot(x, w1e[...]) + _dot(gs_ref[...], w1s[...]) + _dot(gr_ref[...], w1r[...])
    h = jax.nn.relu(h + b1[...])
    h = jax.nn.relu(_dot(h, w2[...]) + b2[...])
    y = _dot(h, w3[...]) + b3[...]
    out_ref[...] = _ln_res(x, y, g[...], bln[...])


def _node_step_body(nl_ref, agg_ref, w1a, w1b, b1, w2, b2, w3, b3, g, bln, out_ref):
    x = nl_ref[...]
    a = agg_ref[0] + agg_ref[1]
    h = jax.nn.relu(_dot(x, w1a[...]) + _dot(a, w1b[...]) + b1[...])
    h = jax.nn.relu(_dot(h, w2[...]) + b2[...])
    y = _dot(h, w3[...]) + b3[...]
    out_ref[...] = _ln_res(x, y, g[...], bln[...])


def _edge_enc_body(fs_ref, fr_ref, w1d, w1nw, w1nm, b1, w2, b2, w3, b3, g, bln,
                   out_ref):
    d = fs_ref[...] - fr_ref[...]
    nw = jnp.sqrt(d[:, 0:1] ** 2 + d[:, 1:2] ** 2 + d[:, 2:3] ** 2)
    nm = jnp.sqrt(d[:, 3:4] ** 2 + d[:, 4:5] ** 2)
    h = _dot(d, w1d[...]) + nw * w1nw[...] + nm * w1nm[...] + b1[...]
    h = jax.nn.relu(h)
    h = jax.nn.relu(_dot(h, w2[...]) + b2[...])
    y = _dot(h, w3[...]) + b3[...]
    mu = jnp.mean(y, axis=-1, keepdims=True)
    var = jnp.mean((y - mu) ** 2, axis=-1, keepdims=True)
    out_ref[...] = (y - mu) * lax.rsqrt(var + 1e-5) * g[...] + bln[...]


def _node_enc_body(nf_ref, w1, b1, w2, b2, w3, b3, g, bln, out_ref):
    h = jax.nn.relu(_dot(nf_ref[...], w1[...]) + b1[...])
    h = jax.nn.relu(_dot(h, w2[...]) + b2[...])
    y = _dot(h, w3[...]) + b3[...]
    mu = jnp.mean(y, axis=-1, keepdims=True)
    var = jnp.mean((y - mu) ** 2, axis=-1, keepdims=True)
    out_ref[...] = (y - mu) * lax.rsqrt(var + 1e-5) * g[...] + bln[...]


def _decoder_body(nl_ref, w1, b1, w2, b2, w3, b3, out_ref):
    h = jax.nn.relu(_dot(nl_ref[...], w1[...]) + b1[...])
    h = jax.nn.relu(_dot(h, w2[...]) + b2[...])
    out_ref[...] = _dot(h, w3[...]) + b3[...]


def _rows(b, d):
    return pl.BlockSpec((b, d), lambda i: (i, 0))


def _whole(shape):
    return pl.BlockSpec(shape, lambda i: tuple(0 for _ in shape))


def _tc_call(body, nrow, brow, row_args, const_args, out_cols=LATENT):
    in_specs = []
    args = []
    for a in row_args:
        if a.ndim == 3:
            in_specs.append(pl.BlockSpec((NC, brow, a.shape[2]), lambda i: (0, i, 0)))
        else:
            in_specs.append(_rows(brow, a.shape[1]))
        args.append(a)
    for a in const_args:
        in_specs.append(_whole(a.shape))
        args.append(a)
    return pl.pallas_call(
        body,
        grid=(nrow // brow,),
        in_specs=in_specs,
        out_specs=_rows(brow, out_cols),
        out_shape=jax.ShapeDtypeStruct((nrow, out_cols), jnp.float32),
    )(*args)


def _b(v):
    return v.reshape(1, -1)


def kernel(world_pos, prev_world_pos, node_type, mesh_pos, cells, params):
    wp = world_pos[0]
    pwp = prev_world_pos[0]
    mp = mesh_pos[0]
    nt = node_type[0]
    c = cells[0]

    e = jnp.concatenate([c[:, 0:2], c[:, 1:3],
                         jnp.stack([c[:, 2], c[:, 0]], axis=-1)], axis=0)
    e = jnp.sort(e, axis=1)
    keys = e[:, 0] * N_NODES + e[:, 1]
    order = jnp.argsort(keys, stable=True)
    e = e[order]
    k = keys[order]
    first = jnp.concatenate([jnp.ones((1,), jnp.bool_), k[1:] != k[:-1]])
    senders = jnp.concatenate([e[:, 0], e[:, 1]]).astype(jnp.int32)
    receivers = jnp.concatenate([e[:, 1], e[:, 0]]).astype(jnp.int32)
    mask = jnp.concatenate([first, first])

    pad = E_PAD - E_DIR
    s_pad = jnp.concatenate([senders, jnp.zeros((pad,), jnp.int32)])
    r_pad = jnp.concatenate([receivers, jnp.zeros((pad,), jnp.int32)])
    rmod = jnp.where(mask, receivers, TRASH).astype(jnp.int32)
    rmod = jnp.concatenate([rmod, jnp.full((pad,), TRASH, jnp.int32)])

    onehot = jax.nn.one_hot(nt[:, 0].astype(jnp.int32), NODE_TYPE_SIZE,
                            dtype=jnp.float32)
    nfeat = jnp.concatenate([wp - pwp, onehot], axis=-1)
    nstd = params["node_norm_std"]
    nmean = params["node_norm_mean"]
    nf16 = jnp.zeros((N_PAD, 16), jnp.float32)
    nf16 = nf16.at[:N_NODES, :12].set((nfeat - nmean) / nstd)

    fpos = jnp.zeros((N_PAD, LATENT), jnp.float32)
    fpos = fpos.at[:N_NODES, 0:3].set(wp).at[:N_NODES, 3:5].set(mp)

    (ew1, eb1), (ew2, eb2), (ew3, eb3) = params["edge_enc"]["mlp"]
    estd = params["edge_norm_std"]
    emean = params["edge_norm_mean"]
    ew1f = ew1 / estd[:, None]
    eb1f = eb1 - (emean / estd) @ ew1
    w1d = jnp.zeros((LATENT, LATENT), jnp.float32)
    w1d = w1d.at[0:3].set(ew1f[0:3]).at[3:5].set(ew1f[4:6])
    w1nw = _b(ew1f[3])
    w1nm = _b(ew1f[6])
    eg, ebn = params["edge_enc"]["ln"]

    (nw1, nb1), (nw2, nb2), (nw3, nb3) = params["node_enc"]["mlp"]
    nw1p = jnp.zeros((16, LATENT), jnp.float32).at[:12].set(nw1)
    ng, nbn = params["node_enc"]["ln"]

    node_lat = _tc_call(
        _node_enc_body, N_PAD, BN, [nf16],
        [nw1p, _b(nb1), nw2, _b(nb2), nw3, _b(nb3), _b(ng), _b(nbn)])

    fs, fr = _gather2_call(fpos, s_pad, r_pad)
    edge_lat = _tc_call(
        _edge_enc_body, E_PAD, BE, [fs, fr],
        [w1d, w1nw, w1nm, _b(eb1f), ew2, _b(eb2), ew3, _b(eb3), _b(eg), _b(ebn)])

    zeros_blk = jnp.zeros((ROWS_PER_SUB, LATENT), jnp.float32)

    for step in params["steps"]:
        (sw1, sb1), (sw2, sb2), (sw3, sb3) = step["edge_mlp"]
        w1e, w1s, w1r = sw1[0:LATENT], sw1[LATENT:2 * LATENT], sw1[2 * LATENT:]
        sg, sbn = step["edge_ln"]
        gs, gr = _gather2_call(node_lat, s_pad, r_pad)
        edge_lat = _tc_call(
            _edge_step_body, E_PAD, BE, [edge_lat, gs, gr],
            [w1e, w1s, w1r, _b(sb1), sw2, _b(sb2), sw3, _b(sb3), _b(sg), _b(sbn)])

        agg2 = _scatter_call(edge_lat, rmod, zeros_blk)

        (tw1, tb1), (tw2, tb2), (tw3, tb3) = step["node_mlp"]
        t1a, t1b = tw1[0:LATENT], tw1[LATENT:]
        tg, tbn = step["node_ln"]
        node_lat = _tc_call(
            _node_step_body, N_PAD, BN, [node_lat, agg2],
            [t1a, t1b, _b(tb1), tw2, _b(tb2), tw3, _b(tb3), _b(tg), _b(tbn)])

    (dw1, db1), (dw2, db2), (dw3, db3) = params["decoder"]
    dw3p = jnp.zeros((LATENT, LATENT), jnp.float32).at[:, :3].set(dw3)
    db3p = jnp.zeros((LATENT,), jnp.float32).at[:3].set(db3)
    dec = _tc_call(
        _decoder_body, N_PAD, BN, [node_lat],
        [dw1, _b(db1), dw2, _b(db2), dw3p, _b(db3p)])
    return dec[:N_NODES, :3]
